# baseline (speedup 1.0000x reference)
import functools

import jax
import jax.numpy as jnp
from jax import lax
from jax.experimental import pallas as pl
from jax.experimental.pallas import tpu as pltpu
from jax.experimental.pallas import tpu_sc as plsc

NF = 26
V = 500
NC = 2
B = 16384
NBIG = NF - 1
W = 2 * 16
BPW = B // W
ICH = 128
NCH = BPW // ICH
ROW_CHUNK = 128
VP = 512
RSTRIDE = NC * VP


def _normalizers_body(*refs):
    big_refs = refs[:NBIG]
    f0_ref, cl_ref, out_ref = refs[NBIG:NBIG + 3]
    flat_refs = refs[NBIG + 3:]
    k = pl.program_id(0)
    nsteps = pl.num_programs(0)

    @pl.when(k == 0)
    def _init():
        out_ref[...] = jnp.zeros((NBIG, NC, V), jnp.float32)

    rows = lax.broadcasted_iota(jnp.int32, (ROW_CHUNK, NC, V), 0) + k * ROW_CHUNK
    valid = rows < V
    for i in range(NBIG):
        blk = big_refs[i][...]
        pad = jnp.zeros((ROW_CHUNK, VP - V), jnp.float32)
        wide = jnp.concatenate(
            [blk[:, 0, :], pad, blk[:, 1, :], pad], axis=1)
        flat_refs[i][...] = wide.reshape(ROW_CHUNK * RSTRIDE)
        e = jnp.where(valid, jnp.exp(blk), 0.0)
        out_ref[i] += jnp.sum(e, axis=0)

    @pl.when(k == nsteps - 1)
    def _finalize():
        small = -jnp.log(out_ref[...])
        f0 = f0_ref[...]
        lse0 = jnp.log(jnp.sum(jnp.exp(f0), axis=1, keepdims=True))
        c0, c1 = cl_ref[0], cl_ref[1]
        z = jnp.log(jnp.exp(c0) + jnp.exp(c1))
        class_row = lax.broadcasted_iota(jnp.int32, (NC, V), 0) == 0
        prior = jnp.where(class_row, c0 - z, c1 - z)
        extra = f0 - lse0 + prior
        row0 = lax.broadcasted_iota(jnp.int32, (NBIG, NC, V), 0) == 0
        out_ref[...] = small + jnp.where(row0, extra[None], 0.0)


def _normalizers_call(bigs_t, f0_t, class_logits):
    grid = (pl.cdiv(V, ROW_CHUNK),)
    in_specs = (
        [pl.BlockSpec((ROW_CHUNK, NC, V), lambda k: (k, 0, 0)) for _ in range(NBIG)]
        + [pl.BlockSpec((NC, V), lambda k: (0, 0)),
           pl.BlockSpec(memory_space=pltpu.SMEM)]
    )
    out_specs = (
        [pl.BlockSpec((NBIG, NC, V), lambda k: (0, 0, 0))]
        + [pl.BlockSpec((ROW_CHUNK * RSTRIDE,), lambda k: (k,))
           for _ in range(NBIG)]
    )
    out_shape = (
        [jax.ShapeDtypeStruct((NBIG, NC, V), jnp.float32)]
        + [jax.ShapeDtypeStruct((V * RSTRIDE,), jnp.float32)
           for _ in range(NBIG)]
    )
    outs = pl.pallas_call(
        _normalizers_body,
        grid=grid,
        in_specs=in_specs,
        out_specs=out_specs,
        out_shape=out_shape,
        compiler_params=pltpu.CompilerParams(
            vmem_limit_bytes=100 * 1024 * 1024),
    )(*bigs_t, f0_t, class_logits)
    return outs[0], outs[1:]


WPF = BPW * NC
NSTR = WPF // ICH


def _gather_body(xt_hbm, small_hbm, *rest):
    big_refs = rest[:NBIG]
    out_hbm = rest[NBIG]
    x_v, small_v, idx_v, dst_v, acc_v, sem = rest[NBIG + 1:]

    wid = lax.axis_index("s") * 2 + lax.axis_index("c")
    base = wid * BPW

    for i in range(NF):
        pltpu.sync_copy(xt_hbm.at[pl.ds(i * B + base, BPW)],
                        x_v.at[pl.ds(i * BPW, BPW)])
    pltpu.sync_copy(small_hbm, small_v)

    lane = lax.iota(jnp.int32, 16)
    half = lane >> 1
    parity = lane & 1

    def idx_body(s, _):
        for i in range(1, NF):
            for k in range(NSTR):
                n0 = k * (ICH // 2) + s * 8 + half
                a = plsc.load_gather(x_v, [n0 + i * BPW])
                b = plsc.load_gather(x_v, [n0 + (i - 1) * BPW])
                idx_v[i - 1, k, pl.ds(s * 16, 16)] = a * RSTRIDE + parity * VP + b
        return 0

    lax.fori_loop(0, ICH // 16, idx_body, 0, unroll=False)

    copies = []
    for i in range(NBIG):
        for k in range(NSTR):
            cp = pltpu.make_async_copy(
                big_refs[i].at[idx_v.at[i, k]],
                dst_v.at[pl.ds(i * WPF + k * ICH, ICH)],
                sem,
            )
            cp.start()
            copies.append(cp)
        if i >= 3:
            for cp in copies[(i - 3) * NSTR:(i - 2) * NSTR]:
                cp.wait()

    def small_body(q, _):
        n = q * 8 + half
        acc = jnp.zeros((16,), jnp.float32)
        for j in range(NBIG):
            xp = plsc.load_gather(x_v, [n + j * BPW])
            acc = acc + plsc.load_gather(small_v, [xp + j * (V * NC) + parity * V])
        acc_v[pl.ds(q * 16, 16)] = acc
        return 0

    lax.fori_loop(0, WPF // 16, small_body, 0, unroll=False)

    for cp in copies[(NBIG - 3) * NSTR:]:
        cp.wait()

    def red_body(q, _):
        acc = acc_v[pl.ds(q * 16, 16)]
        for i in range(NBIG):
            acc = acc + dst_v[pl.ds(i * WPF + q * 16, 16)]
        acc_v[pl.ds(q * 16, 16)] = acc
        return 0

    lax.fori_loop(0, WPF // 16, red_body, 0, unroll=False)

    pltpu.sync_copy(acc_v, out_hbm.at[pl.ds(base * NC, WPF)])


def _gather_call(x_t_flat, small_flat, bigs_flat):
    mesh = plsc.VectorSubcoreMesh(core_axis_name="c", subcore_axis_name="s")
    kern = pl.kernel(
        _gather_body,
        out_type=jax.ShapeDtypeStruct((B * NC,), jnp.float32),
        mesh=mesh,
        scratch_types=[
            pltpu.VMEM((NF * BPW,), jnp.int32),
            pltpu.VMEM((NBIG * V * NC,), jnp.float32),
            pltpu.VMEM((NBIG, NSTR, ICH), jnp.int32),
            pltpu.VMEM((NBIG * WPF,), jnp.float32),
            pltpu.VMEM((WPF,), jnp.float32),
            pltpu.SemaphoreType.DMA,
        ],
        compiler_params=pltpu.CompilerParams(
            use_tc_tiling_on_sc=False, needs_layout_passes=False),
    )
    return kern(x_t_flat, small_flat, *bigs_flat)


@jax.jit
def kernel(x, class_logits, feats):
    bigs_t = [jnp.transpose(f, (0, 2, 1)) for f in feats[1:]]
    f0_t = feats[0].T
    small, bigs_flat = _normalizers_call(bigs_t, f0_t, class_logits)

    x_t_flat = x.T.astype(jnp.int32).reshape(NF * B)
    small_flat = small.reshape(NBIG * V * NC)
    out_flat = _gather_call(x_t_flat, small_flat, bigs_flat)
    return out_flat.reshape(B, NC)


# speedup vs baseline: 1.0003x; 1.0003x over previous
import functools

import jax
import jax.numpy as jnp
from jax import lax
from jax.experimental import pallas as pl
from jax.experimental.pallas import tpu as pltpu
from jax.experimental.pallas import tpu_sc as plsc

NF = 26
V = 500
NC = 2
B = 16384
NBIG = NF - 1
W = 2 * 16
BPW = B // W
ICH = 128
NCH = BPW // ICH
ROW_CHUNK = 64
VP = 512
RSTRIDE = NC * VP


def _normalizers_body(*refs):
    big_refs = refs[:NBIG]
    f0_ref, cl_ref, out_ref = refs[NBIG:NBIG + 3]
    flat_refs = refs[NBIG + 3:]
    k = pl.program_id(0)
    nsteps = pl.num_programs(0)

    @pl.when(k == 0)
    def _init():
        out_ref[...] = jnp.zeros((NBIG, NC, V), jnp.float32)

    rows = lax.broadcasted_iota(jnp.int32, (ROW_CHUNK, NC, V), 0) + k * ROW_CHUNK
    valid = rows < V
    for i in range(NBIG):
        blk = big_refs[i][...]
        pad = jnp.zeros((ROW_CHUNK, VP - V), jnp.float32)
        wide = jnp.concatenate(
            [blk[:, 0, :], pad, blk[:, 1, :], pad], axis=1)
        flat_refs[i][...] = wide.reshape(ROW_CHUNK * RSTRIDE)
        e = jnp.where(valid, jnp.exp(blk), 0.0)
        out_ref[i] += jnp.sum(e, axis=0)

    @pl.when(k == nsteps - 1)
    def _finalize():
        small = -jnp.log(out_ref[...])
        f0 = f0_ref[...]
        lse0 = jnp.log(jnp.sum(jnp.exp(f0), axis=1, keepdims=True))
        c0, c1 = cl_ref[0], cl_ref[1]
        z = jnp.log(jnp.exp(c0) + jnp.exp(c1))
        class_row = lax.broadcasted_iota(jnp.int32, (NC, V), 0) == 0
        prior = jnp.where(class_row, c0 - z, c1 - z)
        extra = f0 - lse0 + prior
        row0 = lax.broadcasted_iota(jnp.int32, (NBIG, NC, V), 0) == 0
        out_ref[...] = small + jnp.where(row0, extra[None], 0.0)


def _normalizers_call(bigs_t, f0_t, class_logits):
    grid = (pl.cdiv(V, ROW_CHUNK),)
    in_specs = (
        [pl.BlockSpec((ROW_CHUNK, NC, V), lambda k: (k, 0, 0)) for _ in range(NBIG)]
        + [pl.BlockSpec((NC, V), lambda k: (0, 0)),
           pl.BlockSpec(memory_space=pltpu.SMEM)]
    )
    out_specs = (
        [pl.BlockSpec((NBIG, NC, V), lambda k: (0, 0, 0))]
        + [pl.BlockSpec((ROW_CHUNK * RSTRIDE,), lambda k: (k,))
           for _ in range(NBIG)]
    )
    out_shape = (
        [jax.ShapeDtypeStruct((NBIG, NC, V), jnp.float32)]
        + [jax.ShapeDtypeStruct((V * RSTRIDE,), jnp.float32)
           for _ in range(NBIG)]
    )
    outs = pl.pallas_call(
        _normalizers_body,
        grid=grid,
        in_specs=in_specs,
        out_specs=out_specs,
        out_shape=out_shape,
        compiler_params=pltpu.CompilerParams(
            vmem_limit_bytes=100 * 1024 * 1024),
    )(*bigs_t, f0_t, class_logits)
    return outs[0], outs[1:]


WPF = BPW * NC
NSTR = WPF // ICH


def _gather_body(xt_hbm, small_hbm, *rest):
    big_refs = rest[:NBIG]
    out_hbm = rest[NBIG]
    x_v, small_v, idx_v, dst_v, acc_v, sem = rest[NBIG + 1:]

    wid = lax.axis_index("s") * 2 + lax.axis_index("c")
    base = wid * BPW

    for i in range(NF):
        pltpu.sync_copy(xt_hbm.at[pl.ds(i * B + base, BPW)],
                        x_v.at[pl.ds(i * BPW, BPW)])
    pltpu.sync_copy(small_hbm, small_v)

    lane = lax.iota(jnp.int32, 16)
    half = lane >> 1
    parity = lane & 1

    def idx_body(s, _):
        for i in range(1, NF):
            for k in range(NSTR):
                n0 = k * (ICH // 2) + s * 8 + half
                a = plsc.load_gather(x_v, [n0 + i * BPW])
                b = plsc.load_gather(x_v, [n0 + (i - 1) * BPW])
                idx_v[i - 1, k, pl.ds(s * 16, 16)] = a * RSTRIDE + parity * VP + b
        return 0

    lax.fori_loop(0, ICH // 16, idx_body, 0, unroll=False)

    copies = []
    for i in range(NBIG):
        for k in range(NSTR):
            cp = pltpu.make_async_copy(
                big_refs[i].at[idx_v.at[i, k]],
                dst_v.at[pl.ds(i * WPF + k * ICH, ICH)],
                sem,
            )
            cp.start()
            copies.append(cp)
        if i >= 3:
            for cp in copies[(i - 3) * NSTR:(i - 2) * NSTR]:
                cp.wait()

    def small_body(q, _):
        n = q * 8 + half
        acc = jnp.zeros((16,), jnp.float32)
        for j in range(NBIG):
            xp = plsc.load_gather(x_v, [n + j * BPW])
            acc = acc + plsc.load_gather(small_v, [xp + j * (V * NC) + parity * V])
        acc_v[pl.ds(q * 16, 16)] = acc
        return 0

    lax.fori_loop(0, WPF // 16, small_body, 0, unroll=False)

    for cp in copies[(NBIG - 3) * NSTR:]:
        cp.wait()

    def red_body(q, _):
        acc = acc_v[pl.ds(q * 16, 16)]
        for i in range(NBIG):
            acc = acc + dst_v[pl.ds(i * WPF + q * 16, 16)]
        acc_v[pl.ds(q * 16, 16)] = acc
        return 0

    lax.fori_loop(0, WPF // 16, red_body, 0, unroll=False)

    pltpu.sync_copy(acc_v, out_hbm.at[pl.ds(base * NC, WPF)])


def _gather_call(x_t_flat, small_flat, bigs_flat):
    mesh = plsc.VectorSubcoreMesh(core_axis_name="c", subcore_axis_name="s")
    kern = pl.kernel(
        _gather_body,
        out_type=jax.ShapeDtypeStruct((B * NC,), jnp.float32),
        mesh=mesh,
        scratch_types=[
            pltpu.VMEM((NF * BPW,), jnp.int32),
            pltpu.VMEM((NBIG * V * NC,), jnp.float32),
            pltpu.VMEM((NBIG, NSTR, ICH), jnp.int32),
            pltpu.VMEM((NBIG * WPF,), jnp.float32),
            pltpu.VMEM((WPF,), jnp.float32),
            pltpu.SemaphoreType.DMA,
        ],
        compiler_params=pltpu.CompilerParams(
            use_tc_tiling_on_sc=False, needs_layout_passes=False),
    )
    return kern(x_t_flat, small_flat, *bigs_flat)


@jax.jit
def kernel(x, class_logits, feats):
    bigs_t = [jnp.transpose(f, (0, 2, 1)) for f in feats[1:]]
    f0_t = feats[0].T
    small, bigs_flat = _normalizers_call(bigs_t, f0_t, class_logits)

    x_t_flat = x.T.astype(jnp.int32).reshape(NF * B)
    small_flat = small.reshape(NBIG * V * NC)
    out_flat = _gather_call(x_t_flat, small_flat, bigs_flat)
    return out_flat.reshape(B, NC)


# speedup vs baseline: 1.1588x; 1.1584x over previous
import jax
import jax.numpy as jnp
from jax import lax
from jax.experimental import pallas as pl
from jax.experimental.pallas import tpu as pltpu
from jax.experimental.pallas import tpu_sc as plsc

NF = 26
V = 500
NC = 2
B = 16384
NBIG = NF - 1
W = 2 * 16
BPW = B // W
ICH = 128
ROW_CHUNK = 64
VP = 512
RSTRIDE = NC * VP
WPF = BPW * NC
NSTR = WPF // ICH
SPLIT = 13


def _make_normalizers_body(n, include_root):
    def body(*refs):
        big_refs = refs[:n]
        f0_ref, cl_ref, out_ref = refs[n:n + 3]
        flat_refs = refs[n + 3:]
        k = pl.program_id(0)
        nsteps = pl.num_programs(0)

        @pl.when(k == 0)
        def _init():
            out_ref[...] = jnp.zeros((n, NC, V), jnp.float32)

        rows = lax.broadcasted_iota(jnp.int32, (ROW_CHUNK, NC, V), 0) + k * ROW_CHUNK
        valid = rows < V
        for i in range(n):
            blk = big_refs[i][...]
            pad = jnp.zeros((ROW_CHUNK, VP - V), jnp.float32)
            wide = jnp.concatenate(
                [blk[:, 0, :], pad, blk[:, 1, :], pad], axis=1)
            flat_refs[i][...] = wide.reshape(ROW_CHUNK * RSTRIDE)
            e = jnp.where(valid, jnp.exp(blk), 0.0)
            out_ref[i] += jnp.sum(e, axis=0)

        @pl.when(k == nsteps - 1)
        def _finalize():
            small = -jnp.log(out_ref[...])
            if include_root:
                f0 = f0_ref[...]
                lse0 = jnp.log(jnp.sum(jnp.exp(f0), axis=1, keepdims=True))
                c0, c1 = cl_ref[0], cl_ref[1]
                z = jnp.log(jnp.exp(c0) + jnp.exp(c1))
                class_row = lax.broadcasted_iota(jnp.int32, (NC, V), 0) == 0
                prior = jnp.where(class_row, c0 - z, c1 - z)
                extra = f0 - lse0 + prior
                row0 = lax.broadcasted_iota(jnp.int32, (n, NC, V), 0) == 0
                small = small + jnp.where(row0, extra[None], 0.0)
            out_ref[...] = small

    return body


def _normalizers_call(bigs_t, f0_t, class_logits, include_root):
    n = len(bigs_t)
    grid = (pl.cdiv(V, ROW_CHUNK),)
    in_specs = (
        [pl.BlockSpec((ROW_CHUNK, NC, V), lambda k: (k, 0, 0)) for _ in range(n)]
        + [pl.BlockSpec((NC, V), lambda k: (0, 0)),
           pl.BlockSpec(memory_space=pltpu.SMEM)]
    )
    out_specs = (
        [pl.BlockSpec((n, NC, V), lambda k: (0, 0, 0))]
        + [pl.BlockSpec((ROW_CHUNK * RSTRIDE,), lambda k: (k,))
           for _ in range(n)]
    )
    out_shape = (
        [jax.ShapeDtypeStruct((n, NC, V), jnp.float32)]
        + [jax.ShapeDtypeStruct((V * RSTRIDE,), jnp.float32)
           for _ in range(n)]
    )
    outs = pl.pallas_call(
        _make_normalizers_body(n, include_root),
        grid=grid,
        in_specs=in_specs,
        out_specs=out_specs,
        out_shape=out_shape,
        compiler_params=pltpu.CompilerParams(
            vmem_limit_bytes=100 * 1024 * 1024),
    )(*bigs_t, f0_t, class_logits)
    return outs[0], outs[1:]


def _make_gather_body(n, jlo):
    nxr = n + 1

    def body(xt_hbm, small_hbm, *rest):
        big_refs = rest[:n]
        out_hbm = rest[n]
        x_v, small_v, idx_v, dst_v, acc_v, sem = rest[n + 1:]

        wid = lax.axis_index("s") * 2 + lax.axis_index("c")
        base = wid * BPW

        for r in range(nxr):
            pltpu.sync_copy(xt_hbm.at[pl.ds((jlo + r) * B + base, BPW)],
                            x_v.at[pl.ds(r * BPW, BPW)])
        pltpu.sync_copy(small_hbm, small_v)

        lane = lax.iota(jnp.int32, 16)
        half = lane >> 1
        parity = lane & 1

        def idx_body(s, _):
            for li in range(n):
                for k in range(NSTR):
                    n0 = k * (ICH // 2) + s * 8 + half
                    a = plsc.load_gather(x_v, [n0 + (li + 1) * BPW])
                    b = plsc.load_gather(x_v, [n0 + li * BPW])
                    idx_v[li, k, pl.ds(s * 16, 16)] = a * RSTRIDE + parity * VP + b
            return 0

        lax.fori_loop(0, ICH // 16, idx_body, 0, unroll=False)

        copies = []
        for li in range(n):
            for k in range(NSTR):
                cp = pltpu.make_async_copy(
                    big_refs[li].at[idx_v.at[li, k]],
                    dst_v.at[pl.ds(li * WPF + k * ICH, ICH)],
                    sem,
                )
                cp.start()
                copies.append(cp)
            if li >= 3:
                for cp in copies[(li - 3) * NSTR:(li - 2) * NSTR]:
                    cp.wait()

        def small_body(q, _):
            pos = q * 8 + half
            acc = jnp.zeros((16,), jnp.float32)
            for li in range(n):
                xp = plsc.load_gather(x_v, [pos + li * BPW])
                acc = acc + plsc.load_gather(
                    small_v, [xp + li * (V * NC) + parity * V])
            acc_v[pl.ds(q * 16, 16)] = acc
            return 0

        lax.fori_loop(0, WPF // 16, small_body, 0, unroll=False)

        for cp in copies[max(n - 3, 0) * NSTR:]:
            cp.wait()

        def red_body(q, _):
            acc = acc_v[pl.ds(q * 16, 16)]
            for li in range(n):
                acc = acc + dst_v[pl.ds(li * WPF + q * 16, 16)]
            acc_v[pl.ds(q * 16, 16)] = acc
            return 0

        lax.fori_loop(0, WPF // 16, red_body, 0, unroll=False)

        pltpu.sync_copy(acc_v, out_hbm.at[pl.ds(base * NC, WPF)])

    return body


def _gather_call(x_t_flat, small_flat, bigs_flat, jlo):
    n = len(bigs_flat)
    mesh = plsc.VectorSubcoreMesh(core_axis_name="c", subcore_axis_name="s")
    kern = pl.kernel(
        _make_gather_body(n, jlo),
        out_type=jax.ShapeDtypeStruct((B * NC,), jnp.float32),
        mesh=mesh,
        scratch_types=[
            pltpu.VMEM(((n + 1) * BPW,), jnp.int32),
            pltpu.VMEM((n * V * NC,), jnp.float32),
            pltpu.VMEM((n, NSTR, ICH), jnp.int32),
            pltpu.VMEM((n * WPF,), jnp.float32),
            pltpu.VMEM((WPF,), jnp.float32),
            pltpu.SemaphoreType.DMA,
        ],
        compiler_params=pltpu.CompilerParams(
            use_tc_tiling_on_sc=False, needs_layout_passes=False),
    )
    return kern(x_t_flat, small_flat, *bigs_flat)


@jax.jit
def kernel(x, class_logits, feats):
    bigs_t = [jnp.transpose(f, (0, 2, 1)) for f in feats[1:]]
    f0_t = feats[0].T
    x_t_flat = x.T.astype(jnp.int32).reshape(NF * B)

    s1, flats1 = _normalizers_call(bigs_t[:SPLIT], f0_t, class_logits, True)
    s2, flats2 = _normalizers_call(bigs_t[SPLIT:], f0_t, class_logits, False)
    out1 = _gather_call(x_t_flat, s1.reshape(-1), flats1, 0)
    out2 = _gather_call(x_t_flat, s2.reshape(-1), flats2, SPLIT)
    return (out1 + out2).reshape(B, NC)


# speedup vs baseline: 1.1862x; 1.0236x over previous
import jax
import jax.numpy as jnp
from jax import lax
from jax.experimental import pallas as pl
from jax.experimental.pallas import tpu as pltpu
from jax.experimental.pallas import tpu_sc as plsc

NF = 26
V = 500
NC = 2
B = 16384
NBIG = NF - 1
W = 2 * 16
BPW = B // W
ICH = 128
ROW_CHUNK = 64
VP = 512
RSTRIDE = NC * VP
WPF = BPW * NC
NSTR = WPF // ICH
GROUPS = (9, 8, 8)


def _make_normalizers_body(n, include_root):
    def body(*refs):
        big_refs = refs[:n]
        f0_ref, cl_ref, out_ref = refs[n:n + 3]
        flat_refs = refs[n + 3:]
        k = pl.program_id(0)
        nsteps = pl.num_programs(0)

        @pl.when(k == 0)
        def _init():
            out_ref[...] = jnp.zeros((n, NC, V), jnp.float32)

        rows = lax.broadcasted_iota(jnp.int32, (ROW_CHUNK, NC, V), 0) + k * ROW_CHUNK
        valid = rows < V
        for i in range(n):
            blk = big_refs[i][...]
            pad = jnp.zeros((ROW_CHUNK, VP - V), jnp.float32)
            wide = jnp.concatenate(
                [blk[:, 0, :], pad, blk[:, 1, :], pad], axis=1)
            flat_refs[i][...] = wide.reshape(ROW_CHUNK * RSTRIDE)
            e = jnp.where(valid, jnp.exp(blk), 0.0)
            out_ref[i] += jnp.sum(e, axis=0)

        @pl.when(k == nsteps - 1)
        def _finalize():
            small = -jnp.log(out_ref[...])
            if include_root:
                f0 = f0_ref[...]
                lse0 = jnp.log(jnp.sum(jnp.exp(f0), axis=1, keepdims=True))
                c0, c1 = cl_ref[0], cl_ref[1]
                z = jnp.log(jnp.exp(c0) + jnp.exp(c1))
                class_row = lax.broadcasted_iota(jnp.int32, (NC, V), 0) == 0
                prior = jnp.where(class_row, c0 - z, c1 - z)
                extra = f0 - lse0 + prior
                row0 = lax.broadcasted_iota(jnp.int32, (n, NC, V), 0) == 0
                small = small + jnp.where(row0, extra[None], 0.0)
            out_ref[...] = small

    return body


def _normalizers_call(bigs_t, f0_t, class_logits, include_root):
    n = len(bigs_t)
    grid = (pl.cdiv(V, ROW_CHUNK),)
    in_specs = (
        [pl.BlockSpec((ROW_CHUNK, NC, V), lambda k: (k, 0, 0)) for _ in range(n)]
        + [pl.BlockSpec((NC, V), lambda k: (0, 0)),
           pl.BlockSpec(memory_space=pltpu.SMEM)]
    )
    out_specs = (
        [pl.BlockSpec((n, NC, V), lambda k: (0, 0, 0))]
        + [pl.BlockSpec((ROW_CHUNK * RSTRIDE,), lambda k: (k,))
           for _ in range(n)]
    )
    out_shape = (
        [jax.ShapeDtypeStruct((n, NC, V), jnp.float32)]
        + [jax.ShapeDtypeStruct((V * RSTRIDE,), jnp.float32)
           for _ in range(n)]
    )
    outs = pl.pallas_call(
        _make_normalizers_body(n, include_root),
        grid=grid,
        in_specs=in_specs,
        out_specs=out_specs,
        out_shape=out_shape,
        compiler_params=pltpu.CompilerParams(
            vmem_limit_bytes=100 * 1024 * 1024),
    )(*bigs_t, f0_t, class_logits)
    return outs[0], outs[1:]


def _make_gather_body(n, jlo):
    nxr = n + 1

    def body(xt_hbm, small_hbm, *rest):
        big_refs = rest[:n]
        out_hbm = rest[n]
        x_v, small_v, idx_v, dst_v, acc_v, sem = rest[n + 1:]

        wid = lax.axis_index("s") * 2 + lax.axis_index("c")
        base = wid * BPW

        for r in range(nxr):
            pltpu.sync_copy(xt_hbm.at[pl.ds((jlo + r) * B + base, BPW)],
                            x_v.at[pl.ds(r * BPW, BPW)])
        pltpu.sync_copy(small_hbm, small_v)

        lane = lax.iota(jnp.int32, 16)
        half = lane >> 1
        parity = lane & 1

        def idx_body(s, _):
            for li in range(n):
                for k in range(NSTR):
                    n0 = k * (ICH // 2) + s * 8 + half
                    a = plsc.load_gather(x_v, [n0 + (li + 1) * BPW])
                    b = plsc.load_gather(x_v, [n0 + li * BPW])
                    idx_v[li, k, pl.ds(s * 16, 16)] = a * RSTRIDE + parity * VP + b
            return 0

        lax.fori_loop(0, ICH // 16, idx_body, 0, unroll=False)

        copies = []
        for li in range(n):
            for k in range(NSTR):
                cp = pltpu.make_async_copy(
                    big_refs[li].at[idx_v.at[li, k]],
                    dst_v.at[pl.ds(li * WPF + k * ICH, ICH)],
                    sem,
                )
                cp.start()
                copies.append(cp)
            if li >= 3:
                for cp in copies[(li - 3) * NSTR:(li - 2) * NSTR]:
                    cp.wait()

        def small_body(q, _):
            pos = q * 8 + half
            acc = jnp.zeros((16,), jnp.float32)
            for li in range(n):
                xp = plsc.load_gather(x_v, [pos + li * BPW])
                acc = acc + plsc.load_gather(
                    small_v, [xp + li * (V * NC) + parity * V])
            acc_v[pl.ds(q * 16, 16)] = acc
            return 0

        lax.fori_loop(0, WPF // 16, small_body, 0, unroll=False)

        for cp in copies[max(n - 3, 0) * NSTR:]:
            cp.wait()

        def red_body(q, _):
            acc = acc_v[pl.ds(q * 16, 16)]
            for li in range(n):
                acc = acc + dst_v[pl.ds(li * WPF + q * 16, 16)]
            acc_v[pl.ds(q * 16, 16)] = acc
            return 0

        lax.fori_loop(0, WPF // 16, red_body, 0, unroll=False)

        pltpu.sync_copy(acc_v, out_hbm.at[pl.ds(base * NC, WPF)])

    return body


def _gather_call(x_t_flat, small_flat, bigs_flat, jlo):
    n = len(bigs_flat)
    mesh = plsc.VectorSubcoreMesh(core_axis_name="c", subcore_axis_name="s")
    kern = pl.kernel(
        _make_gather_body(n, jlo),
        out_type=jax.ShapeDtypeStruct((B * NC,), jnp.float32),
        mesh=mesh,
        scratch_types=[
            pltpu.VMEM(((n + 1) * BPW,), jnp.int32),
            pltpu.VMEM((n * V * NC,), jnp.float32),
            pltpu.VMEM((n, NSTR, ICH), jnp.int32),
            pltpu.VMEM((n * WPF,), jnp.float32),
            pltpu.VMEM((WPF,), jnp.float32),
            pltpu.SemaphoreType.DMA,
        ],
        compiler_params=pltpu.CompilerParams(
            use_tc_tiling_on_sc=False, needs_layout_passes=False),
    )
    return kern(x_t_flat, small_flat, *bigs_flat)


@jax.jit
def kernel(x, class_logits, feats):
    bigs_t = [jnp.transpose(f, (0, 2, 1)) for f in feats[1:]]
    f0_t = feats[0].T
    x_t_flat = x.T.astype(jnp.int32).reshape(NF * B)

    out = None
    jlo = 0
    for g, n in enumerate(GROUPS):
        s, flats = _normalizers_call(
            bigs_t[jlo:jlo + n], f0_t, class_logits, g == 0)
        o = _gather_call(x_t_flat, s.reshape(-1), flats, jlo)
        out = o if out is None else out + o
        jlo += n
    return out.reshape(B, NC)
